# Initial kernel scaffold; baseline (speedup 1.0000x reference)
import jax
import jax.numpy as jnp
from jax.experimental import pallas as pl


def kernel(h, x, v, idxs, params):
    raise NotImplementedError("write your pallas kernel here")


# speedup vs baseline: 5.2701x; 5.2701x over previous
import jax
import jax.numpy as jnp
from jax.experimental import pallas as pl

N = 10000
E = 160000
D = 128
HID = 64
HEADS = 4
RBF = 50
NC = HEADS * HID

EBLK = 2048
EP = ((E + EBLK - 1) // EBLK) * EBLK
NBLK = 2048
NP = ((N + NBLK - 1) // NBLK) * NBLK


def _silu(z):
    return z * jax.nn.sigmoid(z)


def _edge_a(hs_ref, hd_ref, dx_ref,
            win_s, win_d, b_in, means, betas,
            w1_hs, w1_hd, w1_x, w1_n, b1, w2, b2, watt, batt,
            hem_out, att_out, xmx_out):
    hs = hs_ref[...]
    hd = hd_ref[...]
    dx = dx_ref[...]
    d2 = jnp.sum(dx * dx, axis=1, keepdims=True)
    xn = jnp.sqrt(jnp.maximum(d2, 0.0) + 1e-5)
    hh = (jnp.dot(hs, win_s[...], preferred_element_type=jnp.float32)
          + jnp.dot(hd, win_d[...], preferred_element_type=jnp.float32)
          + b_in[...])
    rbf = jnp.exp(-betas[...] * (jnp.exp(-xn) - means[...]) ** 2)
    fx = rbf * hh
    z = (jnp.dot(hs, w1_hs[...], preferred_element_type=jnp.float32)
         + jnp.dot(hd, w1_hd[...], preferred_element_type=jnp.float32)
         + jnp.dot(fx, w1_x[...], preferred_element_type=jnp.float32)
         + xn * w1_n[...]
         + b1[...])
    a1 = _silu(z)
    hem = jnp.dot(a1, w2[...], preferred_element_type=jnp.float32) + b2[...]
    logit = jnp.dot(hem, watt[...], preferred_element_type=jnp.float32) + batt[...]
    att = jnp.where(logit > 0, logit, 2.0 * (jnp.exp(logit / 2.0) - 1.0))
    hem_out[...] = hem
    att_out[...] = att
    xmx_out[...] = dx / (xn + 1e-5)


def _edge_b(hem_ref, catt_ref, xmx_ref, rmat, tmat, wx, wv,
            hea_out, c0_out, c1_out, c2_out, dv_out):
    hem = hem_ref[...]
    catt = catt_ref[...]
    xmx = xmx_ref[...]
    hea = (jnp.dot(hem, rmat[...], preferred_element_type=jnp.float32)
           * jnp.dot(catt, tmat[...], preferred_element_type=jnp.float32))
    coeff = jnp.tanh(jnp.dot(hea, wx[...], preferred_element_type=jnp.float32))
    s = jnp.dot(coeff, wv[...], preferred_element_type=jnp.float32)
    hea_out[...] = hea
    c0_out[...] = coeff * xmx[:, 0:1]
    c1_out[...] = coeff * xmx[:, 1:2]
    c2_out[...] = coeff * xmx[:, 2:3]
    dv_out[...] = xmx * s


def _node_c(h_ref, he_ref, c0_ref, c1_ref, c2_ref, m1_ref, dvs_ref,
            v_ref, x_ref,
            wp1, bp1, wp2, bp2, wn1_h, wn1_e, wn1_c, bn1, wn2, bn2,
            wv1, bv1, wv2,
            hn_out, xn_out, vn_out):
    h = h_ref[...]
    he = he_ref[...]
    m1 = m1_ref[...]
    c0 = c0_ref[...] * m1
    c1 = c1_ref[...] * m1
    c2 = c2_ref[...] * m1
    comb_norm = c0 * c0 + c1 * c1 + c2 * c2
    z = _silu(jnp.dot(comb_norm, wp1[...], preferred_element_type=jnp.float32) + bp1[...])
    h_comb = _silu(jnp.dot(z, wp2[...], preferred_element_type=jnp.float32) + bp2[...])
    z = _silu(jnp.dot(h, wn1_h[...], preferred_element_type=jnp.float32)
              + jnp.dot(he, wn1_e[...], preferred_element_type=jnp.float32)
              + jnp.dot(h_comb, wn1_c[...], preferred_element_type=jnp.float32)
              + bn1[...])
    out = _silu(jnp.dot(z, wn2[...], preferred_element_type=jnp.float32) + bn2[...])
    h_new = h + out
    z = _silu(jnp.dot(h_new, wv1[...], preferred_element_type=jnp.float32) + bv1[...])
    scale = 2.0 * jax.nn.sigmoid(jnp.dot(z, wv2[...], preferred_element_type=jnp.float32))
    dv = dvs_ref[...] * m1
    v_new = dv + scale * v_ref[...]
    hn_out[...] = h_new
    xn_out[...] = x_ref[...] + v_new
    vn_out[...] = v_new


def _pad_rows(a, rows):
    return jnp.concatenate(
        [a, jnp.zeros((rows - a.shape[0],) + a.shape[1:], a.dtype)], axis=0)


def kernel(h, x, v, idxs, params):
    p = params
    src = idxs[:, 0]
    dst = idxs[:, 1]
    hs = _pad_rows(h[src], EP)
    hd = _pad_rows(h[dst], EP)
    dx = _pad_rows(x[src] - x[dst], EP)

    w_in = p['W_in']
    w1 = p['W_eo1']
    eblk = lambda c: pl.BlockSpec((EBLK, c), lambda i: (i, 0))
    full = lambda a: pl.BlockSpec(a.shape, lambda i: (0, 0))
    b_in = p['b_in'].reshape(1, RBF)
    means = p['means'].reshape(1, RBF)
    betas = p['betas'].reshape(1, RBF)
    b1 = p['b_eo1'].reshape(1, HID)
    b2 = p['b_eo2'].reshape(1, HID)
    batt = p['b_att'].reshape(1, HEADS)
    w1_n = w1[2 * D + RBF:].reshape(1, HID)

    wlist_a = [w_in[:D], w_in[D:], b_in, means, betas,
               w1[:D], w1[D:2 * D], w1[2 * D:2 * D + RBF], w1_n, b1,
               p['W_eo2'], b2, p['W_att'], batt]
    hem, att, xmx = pl.pallas_call(
        _edge_a,
        grid=(EP // EBLK,),
        in_specs=[eblk(D), eblk(D), eblk(3)] + [full(a) for a in wlist_a],
        out_specs=[eblk(HID), eblk(HEADS), eblk(3)],
        out_shape=[
            jax.ShapeDtypeStruct((EP, HID), jnp.float32),
            jax.ShapeDtypeStruct((EP, HEADS), jnp.float32),
            jax.ShapeDtypeStruct((EP, 3), jnp.float32),
        ],
    )(hs, hd, dx, *wlist_a)

    att = att[:E]
    seg_max = jax.ops.segment_max(att, src, num_segments=N)
    ex = jnp.exp(att - seg_max[src])
    ssum = jax.ops.segment_sum(ex, src, num_segments=N)
    sem = ex / ssum[src]
    catt = sem / jax.ops.segment_sum(sem, src, num_segments=N)[src]
    catt = _pad_rows(catt, EP)

    rmat = (jnp.arange(NC)[None, :] // HEADS
            == jnp.arange(HID)[:, None]).astype(jnp.float32)
    tmat = (jnp.arange(NC)[None, :] % HEADS
            == jnp.arange(HEADS)[:, None]).astype(jnp.float32)
    wlist_b = [rmat, tmat, p['W_xmix'], p['W_vmix']]
    hea, cw0, cw1, cw2, dv_e = pl.pallas_call(
        _edge_b,
        grid=(EP // EBLK,),
        in_specs=[eblk(HID), eblk(HEADS), eblk(3)] + [full(a) for a in wlist_b],
        out_specs=[eblk(NC), eblk(NC), eblk(NC), eblk(NC), eblk(3)],
        out_shape=[
            jax.ShapeDtypeStruct((EP, NC), jnp.float32),
            jax.ShapeDtypeStruct((EP, NC), jnp.float32),
            jax.ShapeDtypeStruct((EP, NC), jnp.float32),
            jax.ShapeDtypeStruct((EP, NC), jnp.float32),
            jax.ShapeDtypeStruct((EP, 3), jnp.float32),
        ],
    )(hem, catt, xmx, *wlist_b)

    h_e = jax.ops.segment_sum(hea[:E], src, num_segments=N)
    cs0 = jax.ops.segment_sum(cw0[:E], src, num_segments=N)
    cs1 = jax.ops.segment_sum(cw1[:E], src, num_segments=N)
    cs2 = jax.ops.segment_sum(cw2[:E], src, num_segments=N)
    dv_seg = jax.ops.segment_sum(dv_e[:E], src, num_segments=N)
    deg = jax.ops.segment_sum(jnp.ones((E,), jnp.float32), src, num_segments=N)
    m1 = (1.0 / (deg + 1.0)).reshape(N, 1)

    wn1 = p['W_n1']
    nblk = lambda c: pl.BlockSpec((NBLK, c), lambda i: (i, 0))
    bp1 = p['b_pn1'].reshape(1, HID)
    bp2 = p['b_pn2'].reshape(1, HID)
    bn1 = p['b_n1'].reshape(1, HID)
    bn2 = p['b_n2'].reshape(1, D)
    bv1 = p['b_v1'].reshape(1, HID)
    wlist_c = [p['W_pn1'], bp1, p['W_pn2'], bp2,
               wn1[:D], wn1[D:D + NC], wn1[D + NC:], bn1, p['W_n2'], bn2,
               p['W_v1'], bv1, p['W_v2']]
    h_new, x_new, v_new = pl.pallas_call(
        _node_c,
        grid=(NP // NBLK,),
        in_specs=[nblk(D), nblk(NC), nblk(NC), nblk(NC), nblk(NC),
                  nblk(1), nblk(3), nblk(3), nblk(3)]
                 + [full(a) for a in wlist_c],
        out_specs=[nblk(D), nblk(3), nblk(3)],
        out_shape=[
            jax.ShapeDtypeStruct((NP, D), jnp.float32),
            jax.ShapeDtypeStruct((NP, 3), jnp.float32),
            jax.ShapeDtypeStruct((NP, 3), jnp.float32),
        ],
    )(_pad_rows(h, NP), _pad_rows(h_e, NP), _pad_rows(cs0, NP),
      _pad_rows(cs1, NP), _pad_rows(cs2, NP), _pad_rows(m1, NP),
      _pad_rows(dv_seg, NP), _pad_rows(v, NP), _pad_rows(x, NP),
      *wlist_c)

    return (h_new[:N], x_new[:N], v_new[:N])


# speedup vs baseline: 6.9985x; 1.3280x over previous
import jax
import jax.numpy as jnp
from jax.experimental import pallas as pl

N = 10000
E = 160000
D = 128
HID = 64
HEADS = 4
RBF = 50
NC = HEADS * HID

EBLK = 2048
EP = ((E + EBLK - 1) // EBLK) * EBLK
NBLK = 2048
NP = ((N + NBLK - 1) // NBLK) * NBLK


def _silu(z):
    return z * jax.nn.sigmoid(z)


def _edge_a(hs_ref, hd_ref, dx_ref,
            win_s, win_d, b_in, means, betas,
            w1_hs, w1_hd, w1_x, w1_n, b1, w2, b2, watt, batt,
            hem_out, att_out, xmx_out):
    hs = hs_ref[...]
    hd = hd_ref[...]
    dx = dx_ref[...]
    d2 = jnp.sum(dx * dx, axis=1, keepdims=True)
    xn = jnp.sqrt(jnp.maximum(d2, 0.0) + 1e-5)
    hh = (jnp.dot(hs, win_s[...], preferred_element_type=jnp.float32)
          + jnp.dot(hd, win_d[...], preferred_element_type=jnp.float32)
          + b_in[...])
    rbf = jnp.exp(-betas[...] * (jnp.exp(-xn) - means[...]) ** 2)
    fx = rbf * hh
    z = (jnp.dot(hs, w1_hs[...], preferred_element_type=jnp.float32)
         + jnp.dot(hd, w1_hd[...], preferred_element_type=jnp.float32)
         + jnp.dot(fx, w1_x[...], preferred_element_type=jnp.float32)
         + xn * w1_n[...]
         + b1[...])
    a1 = _silu(z)
    hem = jnp.dot(a1, w2[...], preferred_element_type=jnp.float32) + b2[...]
    logit = jnp.dot(hem, watt[...], preferred_element_type=jnp.float32) + batt[...]
    att = jnp.where(logit > 0, logit, 2.0 * (jnp.exp(logit / 2.0) - 1.0))
    hem_out[...] = hem
    att_out[...] = att
    xmx_out[...] = dx / (xn + 1e-5)


def _edge_b(hem_ref, catt_ref, xmx_ref, rmat, tmat, wx, wv,
            big_out, dv_out):
    hem = hem_ref[...]
    catt = catt_ref[...]
    xmx = xmx_ref[...]
    hea = (jnp.dot(hem, rmat[...], preferred_element_type=jnp.float32)
           * jnp.dot(catt, tmat[...], preferred_element_type=jnp.float32))
    coeff = jnp.tanh(jnp.dot(hea, wx[...], preferred_element_type=jnp.float32))
    s = jnp.dot(coeff, wv[...], preferred_element_type=jnp.float32)
    big_out[:, 0:NC] = hea
    big_out[:, NC:2 * NC] = coeff * xmx[:, 0:1]
    big_out[:, 2 * NC:3 * NC] = coeff * xmx[:, 1:2]
    big_out[:, 3 * NC:4 * NC] = coeff * xmx[:, 2:3]
    dv_out[...] = xmx * s


def _node_c(h_ref, he_ref, c0_ref, c1_ref, c2_ref, m1_ref, dvs_ref,
            v_ref, x_ref,
            wp1, bp1, wp2, bp2, wn1_h, wn1_e, wn1_c, bn1, wn2, bn2,
            wv1, bv1, wv2,
            hn_out, xn_out, vn_out):
    h = h_ref[...]
    he = he_ref[...]
    m1 = m1_ref[...]
    c0 = c0_ref[...] * m1
    c1 = c1_ref[...] * m1
    c2 = c2_ref[...] * m1
    comb_norm = c0 * c0 + c1 * c1 + c2 * c2
    z = _silu(jnp.dot(comb_norm, wp1[...], preferred_element_type=jnp.float32) + bp1[...])
    h_comb = _silu(jnp.dot(z, wp2[...], preferred_element_type=jnp.float32) + bp2[...])
    z = _silu(jnp.dot(h, wn1_h[...], preferred_element_type=jnp.float32)
              + jnp.dot(he, wn1_e[...], preferred_element_type=jnp.float32)
              + jnp.dot(h_comb, wn1_c[...], preferred_element_type=jnp.float32)
              + bn1[...])
    out = _silu(jnp.dot(z, wn2[...], preferred_element_type=jnp.float32) + bn2[...])
    h_new = h + out
    z = _silu(jnp.dot(h_new, wv1[...], preferred_element_type=jnp.float32) + bv1[...])
    scale = 2.0 * jax.nn.sigmoid(jnp.dot(z, wv2[...], preferred_element_type=jnp.float32))
    dv = dvs_ref[...] * m1
    v_new = dv + scale * v_ref[...]
    hn_out[...] = h_new
    xn_out[...] = x_ref[...] + v_new
    vn_out[...] = v_new


def _pad_rows(a, rows):
    return jnp.concatenate(
        [a, jnp.zeros((rows - a.shape[0],) + a.shape[1:], a.dtype)], axis=0)


def kernel(h, x, v, idxs, params):
    p = params
    order = jnp.argsort(idxs[:, 0])
    src = idxs[order, 0]
    dst = idxs[order, 1]
    hs = _pad_rows(h[src], EP)
    hd = _pad_rows(h[dst], EP)
    dx = _pad_rows(x[src] - x[dst], EP)

    w_in = p['W_in']
    w1 = p['W_eo1']
    eblk = lambda c: pl.BlockSpec((EBLK, c), lambda i: (i, 0))
    full = lambda a: pl.BlockSpec(a.shape, lambda i: (0, 0))
    b_in = p['b_in'].reshape(1, RBF)
    means = p['means'].reshape(1, RBF)
    betas = p['betas'].reshape(1, RBF)
    b1 = p['b_eo1'].reshape(1, HID)
    b2 = p['b_eo2'].reshape(1, HID)
    batt = p['b_att'].reshape(1, HEADS)
    w1_n = w1[2 * D + RBF:].reshape(1, HID)

    wlist_a = [w_in[:D], w_in[D:], b_in, means, betas,
               w1[:D], w1[D:2 * D], w1[2 * D:2 * D + RBF], w1_n, b1,
               p['W_eo2'], b2, p['W_att'], batt]
    hem, att, xmx = pl.pallas_call(
        _edge_a,
        grid=(EP // EBLK,),
        in_specs=[eblk(D), eblk(D), eblk(3)] + [full(a) for a in wlist_a],
        out_specs=[eblk(HID), eblk(HEADS), eblk(3)],
        out_shape=[
            jax.ShapeDtypeStruct((EP, HID), jnp.float32),
            jax.ShapeDtypeStruct((EP, HEADS), jnp.float32),
            jax.ShapeDtypeStruct((EP, 3), jnp.float32),
        ],
    )(hs, hd, dx, *wlist_a)

    att = att[:E]
    seg_max = jax.ops.segment_max(att, src, num_segments=N,
                                  indices_are_sorted=True)
    ex = jnp.exp(att - seg_max[src])
    ssum = jax.ops.segment_sum(ex, src, num_segments=N,
                               indices_are_sorted=True)
    sem = ex / ssum[src]
    catt = sem / jax.ops.segment_sum(sem, src, num_segments=N,
                                     indices_are_sorted=True)[src]
    catt = _pad_rows(catt, EP)

    rmat = (jnp.arange(NC)[None, :] // HEADS
            == jnp.arange(HID)[:, None]).astype(jnp.float32)
    tmat = (jnp.arange(NC)[None, :] % HEADS
            == jnp.arange(HEADS)[:, None]).astype(jnp.float32)
    wlist_b = [rmat, tmat, p['W_xmix'], p['W_vmix']]
    big, dv_e = pl.pallas_call(
        _edge_b,
        grid=(EP // EBLK,),
        in_specs=[eblk(HID), eblk(HEADS), eblk(3)] + [full(a) for a in wlist_b],
        out_specs=[eblk(4 * NC), eblk(3)],
        out_shape=[
            jax.ShapeDtypeStruct((EP, 4 * NC), jnp.float32),
            jax.ShapeDtypeStruct((EP, 3), jnp.float32),
        ],
    )(hem, catt, xmx, *wlist_b)

    big_seg = jax.ops.segment_sum(big[:E], src, num_segments=N,
                                  indices_are_sorted=True)
    h_e = big_seg[:, 0:NC]
    cs0 = big_seg[:, NC:2 * NC]
    cs1 = big_seg[:, 2 * NC:3 * NC]
    cs2 = big_seg[:, 3 * NC:4 * NC]
    dv_seg = jax.ops.segment_sum(dv_e[:E], src, num_segments=N,
                                 indices_are_sorted=True)
    deg = jax.ops.segment_sum(jnp.ones((E,), jnp.float32), src, num_segments=N,
                              indices_are_sorted=True)
    m1 = (1.0 / (deg + 1.0)).reshape(N, 1)

    wn1 = p['W_n1']
    nblk = lambda c: pl.BlockSpec((NBLK, c), lambda i: (i, 0))
    bp1 = p['b_pn1'].reshape(1, HID)
    bp2 = p['b_pn2'].reshape(1, HID)
    bn1 = p['b_n1'].reshape(1, HID)
    bn2 = p['b_n2'].reshape(1, D)
    bv1 = p['b_v1'].reshape(1, HID)
    wlist_c = [p['W_pn1'], bp1, p['W_pn2'], bp2,
               wn1[:D], wn1[D:D + NC], wn1[D + NC:], bn1, p['W_n2'], bn2,
               p['W_v1'], bv1, p['W_v2']]
    h_new, x_new, v_new = pl.pallas_call(
        _node_c,
        grid=(NP // NBLK,),
        in_specs=[nblk(D), nblk(NC), nblk(NC), nblk(NC), nblk(NC),
                  nblk(1), nblk(3), nblk(3), nblk(3)]
                 + [full(a) for a in wlist_c],
        out_specs=[nblk(D), nblk(3), nblk(3)],
        out_shape=[
            jax.ShapeDtypeStruct((NP, D), jnp.float32),
            jax.ShapeDtypeStruct((NP, 3), jnp.float32),
            jax.ShapeDtypeStruct((NP, 3), jnp.float32),
        ],
    )(_pad_rows(h, NP), _pad_rows(h_e, NP), _pad_rows(cs0, NP),
      _pad_rows(cs1, NP), _pad_rows(cs2, NP), _pad_rows(m1, NP),
      _pad_rows(dv_seg, NP), _pad_rows(v, NP), _pad_rows(x, NP),
      *wlist_c)

    return (h_new[:N], x_new[:N], v_new[:N])
